# baseline (speedup 1.0000x reference)
import functools

import jax
import jax.numpy as jnp
from jax import lax
from jax.experimental import pallas as pl
from jax.experimental.pallas import tpu as pltpu
from jax.experimental.pallas import tpu_sc as plsc

_NC = 2
_NS = 16
_LANES = 16
_TILE = 128
_NBUF = 8


def kernel(x, token_table, pos_table):
    B, S = x.shape
    V, D = token_table.shape
    NW = _NC * _NS
    per_w = (B * S) // NW
    n_tiles = per_w // _TILE

    x_flat = x.reshape(B * S)

    mesh = plsc.VectorSubcoreMesh(core_axis_name="c", subcore_axis_name="s")

    @functools.partial(
        pl.kernel,
        mesh=mesh,
        compiler_params=pltpu.CompilerParams(use_tc_tiling_on_sc=False),
        out_type=jax.ShapeDtypeStruct((B * S, D), jnp.float32),
        scratch_types=(
            [pltpu.VMEM((per_w,), jnp.int32)]
            + [pltpu.VMEM((_TILE, D), jnp.float32)] * _NBUF
            + [pltpu.VMEM((2 * S, D), jnp.float32)]
            + [pltpu.SemaphoreType.DMA] * (2 * _NBUF)
        ),
    )
    def emb_kernel(x_hbm, tok_hbm, pos_hbm, out_hbm, idx_v, *rest):
        rows = rest[:_NBUF]
        pos2x = rest[_NBUF]
        g_sems = rest[_NBUF + 1:2 * _NBUF + 1]
        o_sems = rest[2 * _NBUF + 1:]

        wid = lax.axis_index("s") * _NC + lax.axis_index("c")
        base = wid * per_w

        pltpu.sync_copy(x_hbm.at[pl.ds(base, per_w)], idx_v)
        pltpu.sync_copy(pos_hbm.at[pl.ds(0, S)], pos2x.at[pl.ds(0, S)])
        pltpu.sync_copy(pos_hbm.at[pl.ds(0, S)], pos2x.at[pl.ds(S, S)])

        def gather_tile(t, b):
            pltpu.async_copy(
                tok_hbm.at[idx_v.at[pl.ds(t * _TILE, _TILE)]],
                rows[b], g_sems[b])

        for b in range(_NBUF - 1):
            gather_tile(b, b)

        def outer(i0, carry):
            for b in range(_NBUF):
                t = i0 * _NBUF + b
                b3 = (b + _NBUF - 1) % _NBUF

                pltpu.make_async_copy(
                    out_hbm.at[pl.ds(0, _TILE)], rows[b], g_sems[b]).wait()

                def drain_prev(_b3=b3):
                    pltpu.make_async_copy(
                        rows[_b3], out_hbm.at[pl.ds(0, _TILE)],
                        o_sems[_b3]).wait()

                def issue_next(_t=t, _b3=b3):
                    gather_tile(_t + _NBUF - 1, _b3)

                if b == 0:
                    pl.when(i0 >= 1)(drain_prev)
                    issue_next()
                else:
                    drain_prev()
                    last_issue_i0 = (n_tiles - _NBUF - b) // _NBUF
                    pl.when(i0 <= last_issue_i0)(issue_next)

                s0 = lax.rem(t * _TILE, S)

                def add_body(j, c, _b=b, _s0=s0):
                    for cc in range(D // _LANES):
                        sl = pl.ds(cc * _LANES, _LANES)
                        plsc.addupdate(rows[_b].at[j, sl], pos2x[_s0 + j, sl])
                    return c

                lax.fori_loop(0, _TILE, add_body, 0, unroll=8)

                pltpu.async_copy(
                    rows[b], out_hbm.at[pl.ds(base + t * _TILE, _TILE)],
                    o_sems[b])
            return carry

        lax.fori_loop(0, n_tiles // _NBUF, outer, 0)

        pltpu.make_async_copy(
            rows[_NBUF - 1], out_hbm.at[pl.ds(0, _TILE)],
            o_sems[_NBUF - 1]).wait()

    out = emb_kernel(x_flat, token_table, pos_table)
    return out.reshape(B, S, D)


# speedup vs baseline: 1.0034x; 1.0034x over previous
import functools

import jax
import jax.numpy as jnp
from jax import lax
from jax.experimental import pallas as pl
from jax.experimental.pallas import tpu as pltpu
from jax.experimental.pallas import tpu_sc as plsc

_NC = 2
_NS = 16
_LANES = 16
_TILE = 128
_NBUF = 4


def kernel(x, token_table, pos_table):
    B, S = x.shape
    V, D = token_table.shape
    NW = _NC * _NS
    per_w = (B * S) // NW
    n_tiles = per_w // _TILE

    x_flat = x.reshape(B * S)

    mesh = plsc.VectorSubcoreMesh(core_axis_name="c", subcore_axis_name="s")

    @functools.partial(
        pl.kernel,
        mesh=mesh,
        compiler_params=pltpu.CompilerParams(use_tc_tiling_on_sc=False),
        out_type=jax.ShapeDtypeStruct((B * S, D), jnp.float32),
        scratch_types=(
            [pltpu.VMEM((per_w,), jnp.int32)]
            + [pltpu.VMEM((_TILE, D), jnp.float32)] * _NBUF
            + [pltpu.VMEM((2 * S, D), jnp.float32)]
            + [pltpu.SemaphoreType.DMA] * (2 * _NBUF)
        ),
    )
    def emb_kernel(x_hbm, tok_hbm, pos_hbm, out_hbm, idx_v, *rest):
        rows = rest[:_NBUF]
        pos2x = rest[_NBUF]
        g_sems = rest[_NBUF + 1:2 * _NBUF + 1]
        o_sems = rest[2 * _NBUF + 1:]

        wid = lax.axis_index("s") * _NC + lax.axis_index("c")
        base = wid * per_w

        pltpu.sync_copy(x_hbm.at[pl.ds(base, per_w)], idx_v)
        pltpu.sync_copy(pos_hbm.at[pl.ds(0, S)], pos2x.at[pl.ds(0, S)])
        pltpu.sync_copy(pos_hbm.at[pl.ds(0, S)], pos2x.at[pl.ds(S, S)])

        def gather_tile(t, b):
            pltpu.async_copy(
                tok_hbm.at[idx_v.at[pl.ds(t * _TILE, _TILE)]],
                rows[b], g_sems[b])

        for b in range(_NBUF - 1):
            gather_tile(b, b)

        def outer(i0, carry):
            for b in range(_NBUF):
                t = i0 * _NBUF + b
                b3 = (b + _NBUF - 1) % _NBUF

                pltpu.make_async_copy(
                    out_hbm.at[pl.ds(0, _TILE)], rows[b], g_sems[b]).wait()

                def drain_prev(_b3=b3):
                    pltpu.make_async_copy(
                        rows[_b3], out_hbm.at[pl.ds(0, _TILE)],
                        o_sems[_b3]).wait()

                def issue_next(_t=t, _b3=b3):
                    gather_tile(_t + _NBUF - 1, _b3)

                if b == 0:
                    pl.when(i0 >= 1)(drain_prev)
                    issue_next()
                else:
                    drain_prev()
                    last_issue_i0 = (n_tiles - _NBUF - b) // _NBUF
                    pl.when(i0 <= last_issue_i0)(issue_next)

                s0 = lax.rem(t * _TILE, S)

                def add_body(j, c, _b=b, _s0=s0):
                    for cc in range(D // _LANES):
                        sl = pl.ds(cc * _LANES, _LANES)
                        plsc.addupdate(rows[_b].at[j, sl], pos2x[_s0 + j, sl])
                    return c

                lax.fori_loop(0, _TILE, add_body, 0, unroll=8)

                pltpu.async_copy(
                    rows[b], out_hbm.at[pl.ds(base + t * _TILE, _TILE)],
                    o_sems[b])
            return carry

        lax.fori_loop(0, n_tiles // _NBUF, outer, 0)

        pltpu.make_async_copy(
            rows[_NBUF - 1], out_hbm.at[pl.ds(0, _TILE)],
            o_sems[_NBUF - 1]).wait()

    out = emb_kernel(x_flat, token_table, pos_table)
    return out.reshape(B, S, D)


# speedup vs baseline: 1.0484x; 1.0448x over previous
import functools

import jax
import jax.numpy as jnp
from jax import lax
from jax.experimental import pallas as pl
from jax.experimental.pallas import tpu as pltpu
from jax.experimental.pallas import tpu_sc as plsc

_NC = 2
_NS = 16
_LANES = 16
_TILE = 128
_NBUF = 4


def kernel(x, token_table, pos_table):
    B, S = x.shape
    V, D = token_table.shape
    NW = _NC * _NS
    per_w = (B * S) // NW
    n_tiles = per_w // _TILE

    x_flat = x.reshape(B * S)

    mesh = plsc.VectorSubcoreMesh(core_axis_name="c", subcore_axis_name="s")

    @functools.partial(
        pl.kernel,
        mesh=mesh,
        compiler_params=pltpu.CompilerParams(use_tc_tiling_on_sc=False),
        out_type=jax.ShapeDtypeStruct((B * S, D), jnp.float32),
        scratch_types=(
            [pltpu.VMEM((per_w,), jnp.int32)]
            + [pltpu.VMEM((_TILE, D), jnp.float32)] * _NBUF
            + [pltpu.VMEM((2 * S, D), jnp.float32)]
            + [pltpu.SemaphoreType.DMA] * (2 * _NBUF)
        ),
    )
    def emb_kernel(x_hbm, tok_hbm, pos_hbm, out_hbm, idx_v, *rest):
        rows = rest[:_NBUF]
        pos2x = rest[_NBUF]
        g_sems = rest[_NBUF + 1:2 * _NBUF + 1]
        o_sems = rest[2 * _NBUF + 1:]

        wid = lax.axis_index("s") * _NC + lax.axis_index("c")
        base = wid * per_w

        pltpu.sync_copy(x_hbm.at[pl.ds(base, per_w)], idx_v)
        pltpu.sync_copy(pos_hbm.at[pl.ds(0, S)], pos2x.at[pl.ds(0, S)])
        pltpu.sync_copy(pos_hbm.at[pl.ds(0, S)], pos2x.at[pl.ds(S, S)])

        def gather_tile(t, b):
            pltpu.async_copy(
                tok_hbm.at[idx_v.at[pl.ds(t * _TILE, _TILE)]],
                rows[b], g_sems[b])

        for b in range(_NBUF - 1):
            gather_tile(b, b)

        def outer(i0, carry):
            for b in range(_NBUF):
                t = i0 * _NBUF + b
                b3 = (b + _NBUF - 1) % _NBUF

                pltpu.make_async_copy(
                    out_hbm.at[pl.ds(0, _TILE)], rows[b], g_sems[b]).wait()

                s0 = lax.rem(t * _TILE, S)

                def add_body(j, c, _b=b, _s0=s0):
                    for cc in range(D // _LANES):
                        sl = pl.ds(cc * _LANES, _LANES)
                        plsc.addupdate(rows[_b].at[j, sl], pos2x[_s0 + j, sl])
                    return c

                lax.fori_loop(0, _TILE, add_body, 0, unroll=8)

                def drain_prev(_b3=b3):
                    pltpu.make_async_copy(
                        rows[_b3], out_hbm.at[pl.ds(0, _TILE)],
                        o_sems[_b3]).wait()

                def issue_next(_t=t, _b3=b3):
                    gather_tile(_t + _NBUF - 1, _b3)

                if b == 0:
                    pl.when(i0 >= 1)(drain_prev)
                    issue_next()
                else:
                    drain_prev()
                    last_issue_i0 = (n_tiles - _NBUF - b) // _NBUF
                    pl.when(i0 <= last_issue_i0)(issue_next)

                pltpu.async_copy(
                    rows[b], out_hbm.at[pl.ds(base + t * _TILE, _TILE)],
                    o_sems[b])
            return carry

        lax.fori_loop(0, n_tiles // _NBUF, outer, 0)

        pltpu.make_async_copy(
            rows[_NBUF - 1], out_hbm.at[pl.ds(0, _TILE)],
            o_sems[_NBUF - 1]).wait()

    out = emb_kernel(x_flat, token_table, pos_table)
    return out.reshape(B, S, D)


# speedup vs baseline: 1.0524x; 1.0039x over previous
import functools

import jax
import jax.numpy as jnp
from jax import lax
from jax.experimental import pallas as pl
from jax.experimental.pallas import tpu as pltpu
from jax.experimental.pallas import tpu_sc as plsc

_NC = 2
_NS = 16
_LANES = 16
_TILE = 128
_NBUF = 4


def kernel(x, token_table, pos_table):
    B, S = x.shape
    V, D = token_table.shape
    NW = _NC * _NS
    per_w = (B * S) // NW
    n_tiles = per_w // _TILE

    x_flat = x.reshape(B * S)

    mesh = plsc.VectorSubcoreMesh(core_axis_name="c", subcore_axis_name="s")

    @functools.partial(
        pl.kernel,
        mesh=mesh,
        compiler_params=pltpu.CompilerParams(use_tc_tiling_on_sc=False),
        out_type=jax.ShapeDtypeStruct((B * S, D), jnp.float32),
        scratch_types=(
            [pltpu.VMEM((per_w,), jnp.int32)]
            + [pltpu.VMEM((_TILE, D), jnp.float32)] * _NBUF
            + [pltpu.VMEM((2 * S, D), jnp.float32)]
            + [pltpu.SemaphoreType.DMA] * (2 * _NBUF)
        ),
    )
    def emb_kernel(x_hbm, tok_hbm, pos_hbm, out_hbm, idx_v, *rest):
        rows = rest[:_NBUF]
        pos2x = rest[_NBUF]
        g_sems = rest[_NBUF + 1:2 * _NBUF + 1]
        o_sems = rest[2 * _NBUF + 1:]

        wid = lax.axis_index("s") * _NC + lax.axis_index("c")
        base = wid * per_w

        pltpu.sync_copy(x_hbm.at[pl.ds(base, per_w)], idx_v)
        pltpu.sync_copy(pos_hbm.at[pl.ds(0, S)], pos2x.at[pl.ds(0, S)])
        pltpu.sync_copy(pos_hbm.at[pl.ds(0, S)], pos2x.at[pl.ds(S, S)])

        def gather_tile(t, b):
            pltpu.async_copy(
                tok_hbm.at[idx_v.at[pl.ds(t * _TILE, _TILE)]],
                rows[b], g_sems[b])

        for b in range(_NBUF - 1):
            gather_tile(b, b)

        def outer(i0, carry):
            for b in range(_NBUF):
                t = i0 * _NBUF + b
                b3 = (b + _NBUF - 1) % _NBUF

                pltpu.make_async_copy(
                    out_hbm.at[pl.ds(0, _TILE)], rows[b], g_sems[b]).wait()

                s0 = lax.rem(t * _TILE, S)

                def add_body(j, c, _b=b, _s0=s0):
                    for cc in range(D // _LANES):
                        sl = pl.ds(cc * _LANES, _LANES)
                        plsc.addupdate(rows[_b].at[j, sl], pos2x[_s0 + j, sl])
                    return c

                lax.fori_loop(0, _TILE, add_body, 0, unroll=16)

                def drain_prev(_b3=b3):
                    pltpu.make_async_copy(
                        rows[_b3], out_hbm.at[pl.ds(0, _TILE)],
                        o_sems[_b3]).wait()

                def issue_next(_t=t, _b3=b3):
                    gather_tile(_t + _NBUF - 1, _b3)

                if b == 0:
                    pl.when(i0 >= 1)(drain_prev)
                    issue_next()
                else:
                    drain_prev()
                    last_issue_i0 = (n_tiles - _NBUF - b) // _NBUF
                    pl.when(i0 <= last_issue_i0)(issue_next)

                pltpu.async_copy(
                    rows[b], out_hbm.at[pl.ds(base + t * _TILE, _TILE)],
                    o_sems[b])
            return carry

        lax.fori_loop(0, n_tiles // _NBUF, outer, 0)

        pltpu.make_async_copy(
            rows[_NBUF - 1], out_hbm.at[pl.ds(0, _TILE)],
            o_sems[_NBUF - 1]).wait()

    out = emb_kernel(x_flat, token_table, pos_table)
    return out.reshape(B, S, D)
